# Initial kernel scaffold; baseline (speedup 1.0000x reference)
import functools

import jax
import jax.numpy as jnp
from jax import lax
from jax.experimental import pallas as pl
from jax.experimental.pallas import tpu as pltpu
from jax.experimental.pallas import tpu_sc as plsc

B, T, G, D, V = 4, 1024, 2, 64, 8192
VC = 1024
NVC = V // VC
NB = B * T * G
NC, NS = 2, 16
NW = NC * NS
RPW = NB // NW
ICH = 128
NIC = RPW // ICH


def _argmin_tc_body(xT_ref, c_ref, ids_ref):
    g = pl.program_id(0)
    x = xT_ref[0]

    def body(i, carry):
        bval, bidx = carry
        cb = c_ref[0, pl.ds(i * VC, VC), :]
        c2 = jnp.sum(cb * cb, axis=1, keepdims=True)
        dots = lax.dot_general(cb, x, (((1,), (0,)), ((), ())),
                               preferred_element_type=jnp.float32)
        score = c2 - 2.0 * dots
        cmin = jnp.min(score, axis=0, keepdims=True)
        viota = lax.broadcasted_iota(jnp.int32, score.shape, 0)
        cidx = jnp.min(jnp.where(score == cmin, viota, V),
                       axis=0, keepdims=True) + i * VC
        upd = cmin < bval
        return jnp.where(upd, cmin, bval), jnp.where(upd, cidx, bidx)

    init = (jnp.full((1, T), jnp.inf, jnp.float32),
            jnp.zeros((1, T), jnp.int32))
    _, bidx = lax.fori_loop(0, NVC, body, init)
    ids_ref[0] = bidx + g * V


def _nearest_ids(xT, codebook, interpret=False):
    return pl.pallas_call(
        _argmin_tc_body,
        grid=(G, B),
        in_specs=[
            pl.BlockSpec((1, D, T), lambda g, b: (g, 0, b)),
            pl.BlockSpec((1, V, D), lambda g, b: (g, 0, 0)),
        ],
        out_specs=pl.BlockSpec((1, 1, T), lambda g, b: (g * B + b, 0, 0)),
        out_shape=jax.ShapeDtypeStruct((G * B, 1, T), jnp.int32),
        interpret=interpret,
    )(xT, codebook)


def _sc_gather_body(table_hbm, idx_hbm, mask_hbm, out_hbm,
                    idx_v, mask_v, rows_v, sem):
    wid = lax.axis_index("s") * NC + lax.axis_index("c")
    base = wid * RPW

    pltpu.sync_copy(idx_hbm.at[pl.ds(wid * NIC, NIC)], idx_v)
    pltpu.sync_copy(mask_hbm.at[pl.ds(wid * NIC, NIC)], mask_v)

    copies = []
    for j in range(NIC):
        copies.append(pltpu.make_async_copy(
            table_hbm.at[idx_v.at[j]],
            rows_v.at[pl.ds(j * ICH, ICH)],
            sem,
        ))
    for c in copies:
        c.start()
    for c in copies:
        c.wait()

    def mul_row(r, carry):
        m = plsc.load_gather(
            mask_v,
            [jnp.full((16,), r // ICH, jnp.int32),
             jnp.full((16,), r % ICH, jnp.int32)],
        )
        for c in range(D // 16):
            sl = pl.ds(c * 16, 16)
            rows_v[r, sl] = rows_v[r, sl] * m
        return carry

    lax.fori_loop(0, RPW, mul_row, 0)

    pltpu.sync_copy(rows_v, out_hbm.at[pl.ds(base, RPW)])


@functools.partial(
    pl.kernel,
    mesh=plsc.VectorSubcoreMesh(core_axis_name="c", subcore_axis_name="s"),
    out_type=jax.ShapeDtypeStruct((NB, D), jnp.float32),
    scratch_types=[
        pltpu.VMEM((NIC, ICH), jnp.int32),
        pltpu.VMEM((NIC, ICH), jnp.float32),
        pltpu.VMEM((RPW, D), jnp.float32),
        pltpu.SemaphoreType.DMA,
    ],
)
def _sc_gather(table_hbm, idx_hbm, mask_hbm, out_hbm,
               idx_v, mask_v, rows_v, sem):
    _sc_gather_body(table_hbm, idx_hbm, mask_hbm, out_hbm,
                    idx_v, mask_v, rows_v, sem)


def kernel(inputs, paddings, codebook):
    xT = jnp.transpose(inputs, (2, 3, 0, 1)).reshape(G, D, B * T)
    ids = _nearest_ids(xT, codebook)
    idx_flat = ids.reshape(G, B, T).transpose(1, 2, 0).reshape(NB)
    idx2 = idx_flat.reshape(NB // ICH, ICH)
    mask2 = jnp.repeat(1.0 - paddings.reshape(-1), G).reshape(NB // ICH, ICH)
    table = codebook.reshape(G * V, D)
    out_flat = _sc_gather(table, idx2, mask2)
    return out_flat.reshape(B, T, G, D)


# speedup vs baseline: 1.0021x; 1.0021x over previous
import functools

import jax
import jax.numpy as jnp
from jax import lax
from jax.experimental import pallas as pl
from jax.experimental.pallas import tpu as pltpu
from jax.experimental.pallas import tpu_sc as plsc

B, T, G, D, V = 4, 1024, 2, 64, 8192
VC = 1024
NVC = V // VC
NB = B * T * G
NC, NS = 2, 16
NW = NC * NS
RPW = NB // NW
ICH = 128
NIC = RPW // ICH


def _argmin_tc_body(xT_ref, c_ref, ids_ref):
    g = pl.program_id(0)
    x = xT_ref[0]

    def body(i, carry):
        bval, bidx = carry
        cb = c_ref[0, pl.ds(i * VC, VC), :]
        c2 = jnp.sum(cb * cb, axis=1, keepdims=True)
        dots = lax.dot_general(cb, x, (((1,), (0,)), ((), ())),
                               preferred_element_type=jnp.float32)
        score = c2 - 2.0 * dots
        cmin = jnp.min(score, axis=0, keepdims=True)
        viota = lax.broadcasted_iota(jnp.int32, score.shape, 0)
        cidx = jnp.min(jnp.where(score == cmin, viota, V),
                       axis=0, keepdims=True) + i * VC
        upd = cmin < bval
        return jnp.where(upd, cmin, bval), jnp.where(upd, cidx, bidx)

    init = (jnp.full((1, T), jnp.inf, jnp.float32),
            jnp.zeros((1, T), jnp.int32))
    _, bidx = lax.fori_loop(0, NVC, body, init)
    ids_ref[0] = bidx + g * V


def _nearest_ids(xT, codebook, interpret=False):
    return pl.pallas_call(
        _argmin_tc_body,
        grid=(G, B),
        in_specs=[
            pl.BlockSpec((1, D, T), lambda g, b: (g, 0, b)),
            pl.BlockSpec((1, V, D), lambda g, b: (g, 0, 0)),
        ],
        out_specs=pl.BlockSpec((1, 1, T), lambda g, b: (g * B + b, 0, 0)),
        out_shape=jax.ShapeDtypeStruct((G * B, 1, T), jnp.int32),
        interpret=interpret,
    )(xT, codebook)


def _sc_gather_body(table_hbm, idx_hbm, mask_hbm, out_hbm,
                    idx_v, mask_v, rows_v, sem):
    wid = lax.axis_index("s") * NC + lax.axis_index("c")
    base = wid * RPW

    pltpu.sync_copy(idx_hbm.at[pl.ds(wid * NIC, NIC)], idx_v)
    pltpu.sync_copy(mask_hbm.at[pl.ds(base, RPW)], mask_v)

    copies = []
    for j in range(NIC):
        copies.append(pltpu.make_async_copy(
            table_hbm.at[idx_v.at[j]],
            rows_v.at[pl.ds(j * ICH, ICH)],
            sem,
        ))
    for c in copies:
        c.start()
    for c in copies:
        c.wait()

    def mul_grp(q, carry):
        mv = mask_v[pl.ds(q * 16, 16)]
        for i in range(16):
            m = mv[i]
            r = q * 16 + i
            for c in range(D // 16):
                sl = pl.ds(c * 16, 16)
                rows_v[r, sl] = rows_v[r, sl] * m
        return carry

    lax.fori_loop(0, RPW // 16, mul_grp, 0)

    pltpu.sync_copy(rows_v, out_hbm.at[pl.ds(base, RPW)])


@functools.lru_cache(maxsize=1)
def _sc_gather():
    return pl.kernel(
        _sc_gather_body,
        mesh=plsc.VectorSubcoreMesh(core_axis_name="c", subcore_axis_name="s"),
        out_type=jax.ShapeDtypeStruct((NB, D), jnp.float32),
        scratch_types=[
            pltpu.VMEM((NIC, ICH), jnp.int32),
            pltpu.VMEM((RPW,), jnp.float32),
            pltpu.VMEM((RPW, D), jnp.float32),
            pltpu.SemaphoreType.DMA,
        ],
        compiler_params=pltpu.CompilerParams(use_tc_tiling_on_sc=False),
    )


def kernel(inputs, paddings, codebook):
    xT = jnp.transpose(inputs, (2, 3, 0, 1)).reshape(G, D, B * T)
    ids = _nearest_ids(xT, codebook)
    idx_flat = ids.reshape(G, B, T).transpose(1, 2, 0).reshape(NB)
    idx2 = idx_flat.reshape(NB // ICH, ICH)
    mask2 = jnp.repeat(1.0 - paddings.reshape(-1), G)
    table = codebook.reshape(G * V, D)
    out_flat = _sc_gather()(table, idx2, mask2)
    return out_flat.reshape(B, T, G, D)


# speedup vs baseline: 1.4170x; 1.4140x over previous
import functools

import jax
import jax.numpy as jnp
from jax import lax
from jax.experimental import pallas as pl
from jax.experimental.pallas import tpu as pltpu
from jax.experimental.pallas import tpu_sc as plsc

B, T, G, D, V = 4, 1024, 2, 64, 8192
VC = 1024
NVC = V // VC
NB = B * T * G
NC, NS = 2, 16
NW = NC * NS
RPW = NB // NW
ICH = 128
NIC = RPW // ICH


def _argmin_tc_body(xT_ref, c_ref, ids_ref):
    g = pl.program_id(0)
    x2 = xT_ref[0] * 2.0
    sub_iota = lax.broadcasted_iota(jnp.int32, (8, T), 0).astype(jnp.float32)

    def chunk(ci, carry):
        bval, bidx = carry
        cb = c_ref[0, pl.ds(ci * VC, VC), :]
        c2 = jnp.sum(cb * cb, axis=1, keepdims=True)
        dots2 = lax.dot_general(cb, x2, (((1,), (0,)), ((), ())),
                                preferred_element_type=jnp.float32)

        bval8 = jnp.full((8, T), jnp.inf, jnp.float32)
        bidx8 = jnp.zeros((8, T), jnp.float32)
        for i in range(VC // 8):
            sl = lax.slice(dots2, (i * 8, 0), (i * 8 + 8, T))
            c2s = lax.slice(c2, (i * 8, 0), (i * 8 + 8, 1))
            score = c2s - sl
            m = score < bval8
            bval8 = jnp.minimum(score, bval8)
            bidx8 = jnp.where(m, jnp.float32(i), bidx8)

        cmin = jnp.min(bval8, axis=0, keepdims=True)
        vcand = jnp.where(bval8 == cmin, bidx8 * 8.0 + sub_iota,
                          jnp.float32(V))
        cidx = jnp.min(vcand, axis=0, keepdims=True) + ci * VC
        upd = cmin < bval
        return jnp.where(upd, cmin, bval), jnp.where(upd, cidx, bidx)

    init = (jnp.full((1, T), jnp.inf, jnp.float32),
            jnp.zeros((1, T), jnp.float32))
    _, bidx = lax.fori_loop(0, NVC, chunk, init)
    ids_ref[0] = bidx.astype(jnp.int32) + g * V


def _nearest_ids(xT, codebook, interpret=False):
    return pl.pallas_call(
        _argmin_tc_body,
        grid=(G, B),
        in_specs=[
            pl.BlockSpec((1, D, T), lambda g, b: (g, 0, b)),
            pl.BlockSpec((1, V, D), lambda g, b: (g, 0, 0)),
        ],
        out_specs=pl.BlockSpec((1, 1, T), lambda g, b: (g * B + b, 0, 0)),
        out_shape=jax.ShapeDtypeStruct((G * B, 1, T), jnp.int32),
        interpret=interpret,
    )(xT, codebook)


def _sc_gather_body(table_hbm, idx_hbm, mask_hbm, out_hbm,
                    idx_v, mask_v, rows_v, sem):
    wid = lax.axis_index("s") * NC + lax.axis_index("c")
    base = wid * RPW

    pltpu.sync_copy(idx_hbm.at[pl.ds(wid * NIC, NIC)], idx_v)
    pltpu.sync_copy(mask_hbm.at[pl.ds(base, RPW)], mask_v)

    copies = []
    for j in range(NIC):
        copies.append(pltpu.make_async_copy(
            table_hbm.at[idx_v.at[j]],
            rows_v.at[pl.ds(j * ICH, ICH)],
            sem,
        ))
    for c in copies:
        c.start()
    for c in copies:
        c.wait()

    def mul_grp(q, carry):
        mv = mask_v[pl.ds(q * 16, 16)]
        for i in range(16):
            m = mv[i]
            r = q * 16 + i
            for c in range(D // 16):
                sl = pl.ds(c * 16, 16)
                rows_v[r, sl] = rows_v[r, sl] * m
        return carry

    lax.fori_loop(0, RPW // 16, mul_grp, 0)

    pltpu.sync_copy(rows_v, out_hbm.at[pl.ds(base, RPW)])


@functools.lru_cache(maxsize=1)
def _sc_gather():
    return pl.kernel(
        _sc_gather_body,
        mesh=plsc.VectorSubcoreMesh(core_axis_name="c", subcore_axis_name="s"),
        out_type=jax.ShapeDtypeStruct((NB, D), jnp.float32),
        scratch_types=[
            pltpu.VMEM((NIC, ICH), jnp.int32),
            pltpu.VMEM((RPW,), jnp.float32),
            pltpu.VMEM((RPW, D), jnp.float32),
            pltpu.SemaphoreType.DMA,
        ],
        compiler_params=pltpu.CompilerParams(use_tc_tiling_on_sc=False),
    )


def kernel(inputs, paddings, codebook):
    xT = jnp.transpose(inputs, (2, 3, 0, 1)).reshape(G, D, B * T)
    ids = _nearest_ids(xT, codebook)
    idx_flat = ids.reshape(G, B, T).transpose(1, 2, 0).reshape(NB)
    idx2 = idx_flat.reshape(NB // ICH, ICH)
    mask2 = jnp.repeat(1.0 - paddings.reshape(-1), G)
    table = codebook.reshape(G * V, D)
    out_flat = _sc_gather()(table, idx2, mask2)
    return out_flat.reshape(B, T, G, D)


# speedup vs baseline: 1.5519x; 1.0952x over previous
import functools

import jax
import jax.numpy as jnp
from jax import lax
from jax.experimental import pallas as pl
from jax.experimental.pallas import tpu as pltpu
from jax.experimental.pallas import tpu_sc as plsc

B, T, G, D, V = 4, 1024, 2, 64, 8192
VC = 1024
NVC = V // VC
NB = B * T * G
NC, NS = 2, 16
NW = NC * NS
RPW = NB // NW
ICH = 128
NIC = RPW // ICH


def _argmin_tc_body(xT_ref, c_ref, ids_ref):
    g = pl.program_id(0)
    x2 = xT_ref[0] * 2.0
    sub_iota = lax.broadcasted_iota(jnp.int32, (8, T), 0).astype(jnp.float32)

    def chunk(ci, carry):
        bval, bidx = carry
        cb = c_ref[0, pl.ds(ci * VC, VC), :]
        c2 = jnp.sum(cb * cb, axis=1, keepdims=True)
        dots2 = lax.dot_general(cb, x2, (((1,), (0,)), ((), ())),
                                preferred_element_type=jnp.float32)

        bval8 = jnp.full((8, T), jnp.inf, jnp.float32)
        bidx8 = jnp.zeros((8, T), jnp.float32)
        for i in range(VC // 8):
            sl = lax.slice(dots2, (i * 8, 0), (i * 8 + 8, T))
            c2s = lax.slice(c2, (i * 8, 0), (i * 8 + 8, 1))
            score = c2s - sl
            m = score < bval8
            bval8 = jnp.minimum(score, bval8)
            bidx8 = jnp.where(m, jnp.float32(i), bidx8)

        cmin = jnp.min(bval8, axis=0, keepdims=True)
        vcand = jnp.where(bval8 == cmin, bidx8 * 8.0 + sub_iota,
                          jnp.float32(V))
        cidx = jnp.min(vcand, axis=0, keepdims=True) + ci * VC
        upd = cmin < bval
        return jnp.where(upd, cmin, bval), jnp.where(upd, cidx, bidx)

    carry = (jnp.full((1, T), jnp.inf, jnp.float32),
             jnp.zeros((1, T), jnp.float32))
    for ci in range(NVC):
        carry = chunk(ci, carry)
    _, bidx = carry
    ids_ref[0] = bidx.astype(jnp.int32) + g * V


def _nearest_ids(xT, codebook, interpret=False):
    return pl.pallas_call(
        _argmin_tc_body,
        grid=(G, B),
        in_specs=[
            pl.BlockSpec((1, D, T), lambda g, b: (g, 0, b)),
            pl.BlockSpec((1, V, D), lambda g, b: (g, 0, 0)),
        ],
        out_specs=pl.BlockSpec((1, 1, T), lambda g, b: (g * B + b, 0, 0)),
        out_shape=jax.ShapeDtypeStruct((G * B, 1, T), jnp.int32),
        interpret=interpret,
    )(xT, codebook)


def _sc_gather_body(table_hbm, idx_hbm, mask_hbm, out_hbm,
                    idx_v, mask_v, rows_v, sem):
    wid = lax.axis_index("s") * NC + lax.axis_index("c")
    base = wid * RPW

    pltpu.sync_copy(idx_hbm.at[pl.ds(wid * NIC, NIC)], idx_v)
    pltpu.sync_copy(mask_hbm.at[pl.ds(base, RPW)], mask_v)

    copies = []
    for j in range(NIC):
        copies.append(pltpu.make_async_copy(
            table_hbm.at[idx_v.at[j]],
            rows_v.at[pl.ds(j * ICH, ICH)],
            sem,
        ))
    for c in copies:
        c.start()
    for c in copies:
        c.wait()

    def mul_grp(q, carry):
        mv = mask_v[pl.ds(q * 16, 16)]
        for i in range(16):
            m = mv[i]
            r = q * 16 + i
            for c in range(D // 16):
                sl = pl.ds(c * 16, 16)
                rows_v[r, sl] = rows_v[r, sl] * m
        return carry

    lax.fori_loop(0, RPW // 16, mul_grp, 0)

    pltpu.sync_copy(rows_v, out_hbm.at[pl.ds(base, RPW)])


@functools.lru_cache(maxsize=1)
def _sc_gather():
    return pl.kernel(
        _sc_gather_body,
        mesh=plsc.VectorSubcoreMesh(core_axis_name="c", subcore_axis_name="s"),
        out_type=jax.ShapeDtypeStruct((NB, D), jnp.float32),
        scratch_types=[
            pltpu.VMEM((NIC, ICH), jnp.int32),
            pltpu.VMEM((RPW,), jnp.float32),
            pltpu.VMEM((RPW, D), jnp.float32),
            pltpu.SemaphoreType.DMA,
        ],
        compiler_params=pltpu.CompilerParams(use_tc_tiling_on_sc=False),
    )


def kernel(inputs, paddings, codebook):
    xT = jnp.transpose(inputs, (2, 3, 0, 1)).reshape(G, D, B * T)
    ids = _nearest_ids(xT, codebook)
    idx_flat = ids.reshape(G, B, T).transpose(1, 2, 0).reshape(NB)
    idx2 = idx_flat.reshape(NB // ICH, ICH)
    mask2 = jnp.repeat(1.0 - paddings.reshape(-1), G)
    table = codebook.reshape(G * V, D)
    out_flat = _sc_gather()(table, idx2, mask2)
    return out_flat.reshape(B, T, G, D)


# speedup vs baseline: 1.6116x; 1.0385x over previous
import functools

import jax
import jax.numpy as jnp
from jax import lax
from jax.experimental import pallas as pl
from jax.experimental.pallas import tpu as pltpu
from jax.experimental.pallas import tpu_sc as plsc

B, T, G, D, V = 4, 1024, 2, 64, 8192
VC = 1024
NVC = V // VC
NB = B * T * G
NC, NS = 2, 16
NW = NC * NS
RPW = NB // NW
ICH = 128
NIC = RPW // ICH


def _argmin_tc_body(xT_ref, c_ref, ids_ref):
    g = pl.program_id(0)
    xa = jnp.concatenate([xT_ref[0] * -2.0,
                          jnp.ones((1, T), jnp.float32)], axis=0)
    sub_iota = lax.broadcasted_iota(jnp.int32, (8, T), 0).astype(jnp.float32)

    def chunk(ci, carry):
        bval, bidx = carry
        cb = c_ref[0, pl.ds(ci * VC, VC), :]
        c2 = jnp.sum(cb * cb, axis=1, keepdims=True)
        cba = jnp.concatenate([cb, c2], axis=1)
        scores = lax.dot_general(cba, xa, (((1,), (0,)), ((), ())),
                                 preferred_element_type=jnp.float32)

        bval8 = jnp.full((8, T), jnp.inf, jnp.float32)
        bidx8 = jnp.zeros((8, T), jnp.float32)
        for i in range(VC // 8):
            score = lax.slice(scores, (i * 8, 0), (i * 8 + 8, T))
            m = score < bval8
            bval8 = jnp.minimum(score, bval8)
            bidx8 = jnp.where(m, jnp.float32(i), bidx8)

        cmin = jnp.min(bval8, axis=0, keepdims=True)
        vcand = jnp.where(bval8 == cmin, bidx8 * 8.0 + sub_iota,
                          jnp.float32(V))
        cidx = jnp.min(vcand, axis=0, keepdims=True) + ci * VC
        upd = cmin < bval
        return jnp.where(upd, cmin, bval), jnp.where(upd, cidx, bidx)

    carry = (jnp.full((1, T), jnp.inf, jnp.float32),
             jnp.zeros((1, T), jnp.float32))
    for ci in range(NVC):
        carry = chunk(ci, carry)
    _, bidx = carry
    ids_ref[0] = bidx.astype(jnp.int32) + g * V


def _nearest_ids(xT, codebook, interpret=False):
    return pl.pallas_call(
        _argmin_tc_body,
        grid=(G, B),
        in_specs=[
            pl.BlockSpec((1, D, T), lambda g, b: (g, 0, b)),
            pl.BlockSpec((1, V, D), lambda g, b: (g, 0, 0)),
        ],
        out_specs=pl.BlockSpec((1, 1, T), lambda g, b: (g * B + b, 0, 0)),
        out_shape=jax.ShapeDtypeStruct((G * B, 1, T), jnp.int32),
        interpret=interpret,
    )(xT, codebook)


def _sc_gather_body(table_hbm, idx_hbm, mask_hbm, out_hbm,
                    idx_v, mask_v, rows_v, sem):
    wid = lax.axis_index("s") * NC + lax.axis_index("c")
    base = wid * RPW

    pltpu.sync_copy(idx_hbm.at[pl.ds(wid * NIC, NIC)], idx_v)
    pltpu.sync_copy(mask_hbm.at[pl.ds(base, RPW)], mask_v)

    copies = []
    for j in range(NIC):
        copies.append(pltpu.make_async_copy(
            table_hbm.at[idx_v.at[j]],
            rows_v.at[pl.ds(j * ICH, ICH)],
            sem,
        ))
    for c in copies:
        c.start()
    for c in copies:
        c.wait()

    def mul_grp(q, carry):
        mv = mask_v[pl.ds(q * 16, 16)]
        for i in range(16):
            m = mv[i]
            r = q * 16 + i
            for c in range(D // 16):
                sl = pl.ds(c * 16, 16)
                rows_v[r, sl] = rows_v[r, sl] * m
        return carry

    lax.fori_loop(0, RPW // 16, mul_grp, 0)

    pltpu.sync_copy(rows_v, out_hbm.at[pl.ds(base, RPW)])


@functools.lru_cache(maxsize=1)
def _sc_gather():
    return pl.kernel(
        _sc_gather_body,
        mesh=plsc.VectorSubcoreMesh(core_axis_name="c", subcore_axis_name="s"),
        out_type=jax.ShapeDtypeStruct((NB, D), jnp.float32),
        scratch_types=[
            pltpu.VMEM((NIC, ICH), jnp.int32),
            pltpu.VMEM((RPW,), jnp.float32),
            pltpu.VMEM((RPW, D), jnp.float32),
            pltpu.SemaphoreType.DMA,
        ],
        compiler_params=pltpu.CompilerParams(use_tc_tiling_on_sc=False),
    )


def kernel(inputs, paddings, codebook):
    xT = jnp.transpose(inputs, (2, 3, 0, 1)).reshape(G, D, B * T)
    ids = _nearest_ids(xT, codebook)
    idx_flat = ids.reshape(G, B, T).transpose(1, 2, 0).reshape(NB)
    idx2 = idx_flat.reshape(NB // ICH, ICH)
    mask2 = jnp.repeat(1.0 - paddings.reshape(-1), G)
    table = codebook.reshape(G * V, D)
    out_flat = _sc_gather()(table, idx2, mask2)
    return out_flat.reshape(B, T, G, D)
